# baseline (speedup 1.0000x reference)
import functools

import numpy as np
import jax
import jax.numpy as jnp
from jax import lax
from jax.experimental import pallas as pl
from jax.experimental.pallas import tpu as pltpu
from jax.experimental.pallas import tpu_sc as plsc

H = 224
HH = H // 2
OW = 4 * H
NPLANES = 4 * 192
PLANE = H * H
OUT_PLANE = HH * OW
NWORKERS = 32
PER_WORKER = NPLANES // NWORKERS
NCHUNK = 8
CHUNK = OUT_PLANE // NCHUNK
PVREG = CHUNK // 32


def _build_src_map() -> np.ndarray:
    lmid = (H - 1) // 2
    r = np.arange(HH)[:, None]
    j = np.arange(OW)[None, :]
    i = lmid - r
    el = 2 * r + 1
    hi = i + el
    b1 = 3 * i + el
    b2 = 3 * i + 2 * el
    b3 = 7 * i + 3 * el
    b4 = 7 * i + 4 * el
    body = 4 * H - 4
    k = 5 * i + 2 * el + hi
    src_a = i * H + np.clip(j - body * (j >= b4), i, hi)
    src_b = hi * H + np.clip(k - j, i, hi)
    src_cr = (j - (2 * i + el)) * H + hi
    src_cl = (body - j) * H + i
    src = np.where(j < b1, src_a,
          np.where(j < b2, src_cr,
          np.where(j < b3, src_b,
          np.where(j < b4, src_cl, src_a))))
    return src.reshape(-1)


def _build_packed_idx() -> np.ndarray:
    flat = _build_src_map().astype(np.uint32).reshape(-1, 2, 16)
    packed = flat[:, 0, :] | (flat[:, 1, :] << 16)
    return packed.reshape(-1).view(np.int32)


_IDX_PACKED = _build_packed_idx()

def _sqrl_gather_body(x_hbm, idx_hbm, out_hbm, idx_v, plane_v, outb_v,
                      insem, osem0, osem1):
    wid = lax.axis_index("s") * 2 + lax.axis_index("c")
    osems = (osem0, osem1)
    pltpu.sync_copy(idx_hbm, idx_v)

    def plane_body(pi, carry):
        p = wid * PER_WORKER + pi
        pltpu.async_copy(x_hbm.at[p], plane_v, insem).wait()
        handles = []
        for c in range(NCHUNK):
            if c >= 2:
                handles[c - 2].wait()

            @plsc.parallel_loop(0, PVREG, unroll=16)
            def vbody(k, c=c):
                vp = idx_v[pl.ds(c * (PVREG * 16) + k * 16, 16)]
                lo = jnp.bitwise_and(vp, 0xFFFF)
                hi2 = lax.shift_right_logical(vp, 16)
                outb_v[c % 2, pl.ds(k * 32, 16)] = plsc.load_gather(plane_v, [lo])
                outb_v[c % 2, pl.ds(k * 32 + 16, 16)] = plsc.load_gather(plane_v, [hi2])
            handles.append(pltpu.async_copy(
                outb_v.at[c % 2],
                out_hbm.at[p, pl.ds(c * CHUNK, CHUNK)],
                osems[c % 2]))
        handles[-2].wait()
        handles[-1].wait()
        return carry

    lax.fori_loop(0, PER_WORKER, plane_body, 0)


@functools.cache
def _sqrl_gather():
    mesh = plsc.VectorSubcoreMesh(core_axis_name="c", subcore_axis_name="s")
    return pl.kernel(
        _sqrl_gather_body,
        out_type=jax.ShapeDtypeStruct((NPLANES, OUT_PLANE), jnp.float32),
        mesh=mesh,
        scratch_types=[
            pltpu.VMEM((PLANE,), jnp.int32),
            pltpu.VMEM((PLANE,), jnp.float32),
            pltpu.VMEM((2, CHUNK), jnp.float32),
            pltpu.SemaphoreType.DMA,
            pltpu.SemaphoreType.DMA,
            pltpu.SemaphoreType.DMA,
        ],
        compiler_params=pltpu.CompilerParams(needs_layout_passes=False),
    )


def kernel(x):
    b, ch, h, w = x.shape
    x2 = x.reshape(NPLANES, PLANE)
    out2 = _sqrl_gather()(x2, jnp.asarray(_IDX_PACKED))
    return out2.reshape(b, ch, HH, OW)


# speedup vs baseline: 1.0215x; 1.0215x over previous
import functools

import numpy as np
import jax
import jax.numpy as jnp
from jax import lax
from jax.experimental import pallas as pl
from jax.experimental.pallas import tpu as pltpu
from jax.experimental.pallas import tpu_sc as plsc

H = 224
HH = H // 2
OW = 4 * H
NPLANES = 4 * 192
PLANE = H * H
OUT_PLANE = HH * OW
NWORKERS = 32
PER_WORKER = NPLANES // NWORKERS
NCHUNK = 8
CHUNK = OUT_PLANE // NCHUNK
PVREG = CHUNK // 32


def _build_src_map() -> np.ndarray:
    lmid = (H - 1) // 2
    r = np.arange(HH)[:, None]
    j = np.arange(OW)[None, :]
    i = lmid - r
    el = 2 * r + 1
    hi = i + el
    b1 = 3 * i + el
    b2 = 3 * i + 2 * el
    b3 = 7 * i + 3 * el
    b4 = 7 * i + 4 * el
    body = 4 * H - 4
    k = 5 * i + 2 * el + hi
    src_a = i * H + np.clip(j - body * (j >= b4), i, hi)
    src_b = hi * H + np.clip(k - j, i, hi)
    src_cr = (j - (2 * i + el)) * H + hi
    src_cl = (body - j) * H + i
    src = np.where(j < b1, src_a,
          np.where(j < b2, src_cr,
          np.where(j < b3, src_b,
          np.where(j < b4, src_cl, src_a))))
    return src.reshape(-1)


def _build_packed_idx() -> np.ndarray:
    flat = _build_src_map().astype(np.uint32).reshape(-1, 2, 16)
    packed = flat[:, 0, :] | (flat[:, 1, :] << 16)
    return packed.reshape(-1).view(np.int32)


_IDX_PACKED = _build_packed_idx()

def _sqrl_gather_body(x_hbm, idx_hbm, out_hbm, idx_v, plane_v, outb_v,
                      insem, osem0, osem1):
    wid = lax.axis_index("s") * 2 + lax.axis_index("c")
    osems = (osem0, osem1)
    pltpu.sync_copy(idx_hbm, idx_v)

    def plane_body(pi, carry):
        p = wid * PER_WORKER + pi
        pltpu.async_copy(x_hbm.at[p], plane_v, insem).wait()
        handles = []
        for c in range(NCHUNK):
            if c >= 2:
                handles[c - 2].wait()

            @plsc.parallel_loop(0, PVREG, unroll=8)
            def vbody(k, c=c):
                vp = idx_v[pl.ds(c * (PVREG * 16) + k * 16, 16)]
                lo = jnp.bitwise_and(vp, 0xFFFF)
                hi2 = lax.shift_right_logical(vp, 16)
                outb_v[c % 2, pl.ds(k * 32, 16)] = plsc.load_gather(plane_v, [lo])
                outb_v[c % 2, pl.ds(k * 32 + 16, 16)] = plsc.load_gather(plane_v, [hi2])
            handles.append(pltpu.async_copy(
                outb_v.at[c % 2],
                out_hbm.at[p, pl.ds(c * CHUNK, CHUNK)],
                osems[c % 2]))
        handles[-2].wait()
        handles[-1].wait()
        return carry

    lax.fori_loop(0, PER_WORKER, plane_body, 0)


@functools.cache
def _sqrl_gather():
    mesh = plsc.VectorSubcoreMesh(core_axis_name="c", subcore_axis_name="s")
    return pl.kernel(
        _sqrl_gather_body,
        out_type=jax.ShapeDtypeStruct((NPLANES, OUT_PLANE), jnp.float32),
        mesh=mesh,
        scratch_types=[
            pltpu.VMEM((PLANE,), jnp.int32),
            pltpu.VMEM((PLANE,), jnp.float32),
            pltpu.VMEM((2, CHUNK), jnp.float32),
            pltpu.SemaphoreType.DMA,
            pltpu.SemaphoreType.DMA,
            pltpu.SemaphoreType.DMA,
        ],
        compiler_params=pltpu.CompilerParams(needs_layout_passes=False),
    )


def kernel(x):
    b, ch, h, w = x.shape
    x2 = x.reshape(NPLANES, PLANE)
    out2 = _sqrl_gather()(x2, jnp.asarray(_IDX_PACKED))
    return out2.reshape(b, ch, HH, OW)


# speedup vs baseline: 1.3408x; 1.3126x over previous
import functools

import numpy as np
import jax
import jax.numpy as jnp
from jax import lax
from jax.experimental import pallas as pl
from jax.experimental.pallas import tpu as pltpu
from jax.experimental.pallas import tpu_sc as plsc

H = 224
HH = H // 2
OW = 4 * H
B = 4
C = 192
NPLANES = B * C
OUT_PLANE = HH * OW
NWORKERS = 32
PER_WORKER = NPLANES // NWORKERS
CROWS = 8
NCHUNK = HH // CROWS
CHUNK = CROWS * OW
ROWVREG = OW // 32
IDXWORDS = OUT_PLANE // 2


def _build_src_map() -> np.ndarray:
    lmid = (H - 1) // 2
    r = np.arange(HH)[:, None]
    j = np.arange(OW)[None, :]
    i = lmid - r
    el = 2 * r + 1
    hi = i + el
    b1 = 3 * i + el
    b2 = 3 * i + 2 * el
    b3 = 7 * i + 3 * el
    b4 = 7 * i + 4 * el
    body = 4 * H - 4
    k = 5 * i + 2 * el + hi
    src_a = i * H + np.clip(j - body * (j >= b4), i, hi)
    src_b = hi * H + np.clip(k - j, i, hi)
    src_cr = (j - (2 * i + el)) * H + hi
    src_cl = (body - j) * H + i
    src = np.where(j < b1, src_a,
          np.where(j < b2, src_cr,
          np.where(j < b3, src_b,
          np.where(j < b4, src_cl, src_a))))
    return src.reshape(-1)


def _build_packed_idx() -> np.ndarray:
    flat = _build_src_map().astype(np.uint32).reshape(-1, 2, 16)
    r0, c0 = flat[:, 0, :] // H, flat[:, 0, :] % H
    r1, c1 = flat[:, 1, :] // H, flat[:, 1, :] % H
    packed = r0 | (c0 << 8) | (r1 << 16) | (c1 << 24)
    return packed.reshape(-1).view(np.int32)


_IDX_PACKED = _build_packed_idx()


def _sqrl_gather_body(x_hbm, idx_hbm, out_hbm, idx_v, plane_v, outb_v,
                      insem, osem):
    wid = lax.axis_index("s") * 2 + lax.axis_index("c")
    pltpu.sync_copy(idx_hbm, idx_v)

    def drain_chunk(buf):
        pltpu.make_async_copy(
            out_hbm.at[0, 0, pl.ds(0, CROWS), :], outb_v.at[buf], osem.at[buf]
        ).wait()

    def plane_body(pi, carry):
        p = wid * PER_WORKER + pi
        pb = lax.div(p, C)
        pc = lax.rem(p, C)
        pltpu.async_copy(x_hbm.at[pb, pc], plane_v, insem).wait()

        def chunk_body(c, carry2):
            buf = lax.rem(c, 2)

            @pl.when(c >= 2)
            def _():
                drain_chunk(buf)

            @plsc.parallel_loop(0, CROWS, unroll=1)
            def vbody(row):
                base = (c * (CROWS * ROWVREG) + row * ROWVREG) * 16
                for kk in range(ROWVREG):
                    vp = idx_v[pl.ds(base + kk * 16, 16)]
                    r0 = jnp.bitwise_and(vp, 0xFF)
                    c0 = jnp.bitwise_and(lax.shift_right_logical(vp, 8), 0xFF)
                    r1 = jnp.bitwise_and(lax.shift_right_logical(vp, 16), 0xFF)
                    c1 = lax.shift_right_logical(vp, 24)
                    outb_v[buf, row, pl.ds(kk * 32, 16)] = (
                        plsc.load_gather(plane_v, [r0, c0]))
                    outb_v[buf, row, pl.ds(kk * 32 + 16, 16)] = (
                        plsc.load_gather(plane_v, [r1, c1]))

            pltpu.async_copy(
                outb_v.at[buf],
                out_hbm.at[pb, pc, pl.ds(c * CROWS, CROWS), :],
                osem.at[buf])
            return carry2

        lax.fori_loop(0, NCHUNK, chunk_body, 0)
        drain_chunk(0)
        drain_chunk(1)
        return carry

    lax.fori_loop(0, PER_WORKER, plane_body, 0)


@functools.cache
def _sqrl_gather():
    mesh = plsc.VectorSubcoreMesh(core_axis_name="c", subcore_axis_name="s")
    return pl.kernel(
        _sqrl_gather_body,
        out_type=jax.ShapeDtypeStruct((B, C, HH, OW), jnp.float32),
        mesh=mesh,
        scratch_types=[
            pltpu.VMEM((IDXWORDS,), jnp.int32),
            pltpu.VMEM((H, H), jnp.float32),
            pltpu.VMEM((2, CROWS, OW), jnp.float32),
            pltpu.SemaphoreType.DMA,
            pltpu.SemaphoreType.DMA((2,)),
        ],
        compiler_params=pltpu.CompilerParams(needs_layout_passes=False),
    )


def kernel(x):
    return _sqrl_gather()(x, jnp.asarray(_IDX_PACKED))
